# baseline (speedup 1.0000x reference)
import jax
import jax.numpy as jnp
from jax import lax
from jax.experimental import pallas as pl
from jax.experimental.pallas import tpu as pltpu
from jax.experimental.pallas import tpu_sc as plsc

NC = 2
NS = 16
NW = NC * NS
L = 16

EMBED = 768
R_TOTAL = 1024 * 128
ROWS_PER_W = R_TOTAL // NW
CHUNK = 16
N_CHUNKS = ROWS_PER_W // CHUNK
VREGS_PER_ROW = EMBED // L


def _sc_kernel(idx_hbm, comb_tab, out_hbm,
               idx_v, g0, w0, g1, w1, sem_g0, sem_g1, sem_w):
    wid = lax.axis_index("s") * NC + lax.axis_index("c")
    w_base = wid * ROWS_PER_W

    gb = (g0, g1)
    wb = (w0, w1)
    sem_g = (sem_g0, sem_g1)

    pltpu.sync_copy(idx_hbm.at[pl.ds(w_base * 3, ROWS_PER_W * 3)], idx_v)

    def gather_desc(c, b):
        s = pl.ds(c * (3 * CHUNK), 3 * CHUNK)
        return pltpu.make_async_copy(comb_tab.at[idx_v.at[s]], gb[b],
                                     sem_g[b])

    def wb_desc(c, b):
        return pltpu.make_async_copy(
            wb[b], out_hbm.at[pl.ds(w_base + c * CHUNK, CHUNK)], sem_w)

    gather_desc(0, 0).start()

    @pl.loop(0, N_CHUNKS, step=2)
    def _pair(cc):
        for b in range(2):
            c = cc + b
            nb = 1 - b
            if b == 0:
                @pl.when(cc > 0)
                def _():
                    wb_desc(cc - 1, nb).wait()
                gather_desc(c + 1, nb).start()
            else:
                wb_desc(c - 1, nb).wait()

                @pl.when(cc < N_CHUNKS - 2)
                def _():
                    gather_desc(c + 1, nb).start()
            gather_desc(c, b).wait()

            @pl.loop(0, CHUNK)
            def _row(r):
                for k in range(VREGS_PER_ROW):
                    sl = pl.ds(k * L, L)
                    wb[b][r, sl] = (gb[b][3 * r, sl] + gb[b][3 * r + 1, sl]
                                    + gb[b][3 * r + 2, sl])

            wb_desc(c, b).start()

    wb_desc(N_CHUNKS - 1, 1).wait()


@jax.jit
def _run(idx_comb, comb_tab):
    mesh = plsc.VectorSubcoreMesh(
        core_axis_name="c", subcore_axis_name="s", num_cores=NC,
        num_subcores=NS)
    f = pl.kernel(
        _sc_kernel,
        out_type=jax.ShapeDtypeStruct((R_TOTAL, EMBED), jnp.float32),
        mesh=mesh,
        scratch_types=[
            pltpu.VMEM((3 * ROWS_PER_W,), jnp.int32),
            pltpu.VMEM((3 * CHUNK, EMBED), jnp.float32),
            pltpu.VMEM((CHUNK, EMBED), jnp.float32),
            pltpu.VMEM((3 * CHUNK, EMBED), jnp.float32),
            pltpu.VMEM((CHUNK, EMBED), jnp.float32),
            pltpu.SemaphoreType.DMA,
            pltpu.SemaphoreType.DMA,
            pltpu.SemaphoreType.DMA,
        ],
    )
    return f(idx_comb, comb_tab)


def kernel(node_type, in_degree, out_degree, node_table, in_degree_table,
           out_degree_table):
    n_graph, n_node = in_degree.shape
    n_node_rows = node_table.shape[0]
    n_in_rows = in_degree_table.shape[0]
    nt = node_type.reshape(-1).astype(jnp.int32)
    ind = in_degree.reshape(-1).astype(jnp.int32) + n_node_rows
    outd = out_degree.reshape(-1).astype(jnp.int32) + n_node_rows + n_in_rows
    idx_comb = jnp.stack([nt, ind, outd], axis=-1).reshape(-1)
    comb_tab = jnp.concatenate(
        [node_table, in_degree_table, out_degree_table], axis=0)
    out = _run(idx_comb, comb_tab)
    return out.reshape(n_graph, n_node, EMBED)


# speedup vs baseline: 1.8413x; 1.8413x over previous
import jax
import jax.numpy as jnp
from jax import lax
from jax.experimental import pallas as pl
from jax.experimental.pallas import tpu as pltpu
from jax.experimental.pallas import tpu_sc as plsc

NC = 2
NS = 16
NW = NC * NS
L = 16

EMBED = 768
R_TOTAL = 1024 * 128
ROWS_PER_W = R_TOTAL // NW
CHUNK = 8
NSETS = 4
N_CHUNKS = ROWS_PER_W // CHUNK
VREGS_PER_ROW = EMBED // L


def _sc_kernel(nt_hbm, in_hbm, ot_hbm, node_tab, in_tab, out_tab, out_hbm,
               idx_n, idx_i, idx_o, bufs_and_sems):
    (bn0, bi0, bo0, bn1, bi1, bo1, bn2, bi2, bo2, bn3, bi3, bo3,
     sg0, sg1, sg2, sg3, sem_w) = bufs_and_sems
    wid = lax.axis_index("s") * NC + lax.axis_index("c")
    w_base = wid * ROWS_PER_W

    bn = (bn0, bn1, bn2, bn3)
    bi = (bi0, bi1, bi2, bi3)
    bo = (bo0, bo1, bo2, bo3)
    sem_g = (sg0, sg1, sg2, sg3)

    pltpu.sync_copy(nt_hbm.at[pl.ds(w_base, ROWS_PER_W)], idx_n)
    pltpu.sync_copy(in_hbm.at[pl.ds(w_base, ROWS_PER_W)], idx_i)
    pltpu.sync_copy(ot_hbm.at[pl.ds(w_base, ROWS_PER_W)], idx_o)

    def gather_descs(c, b):
        s = pl.ds(c * CHUNK, CHUNK)
        return (
            pltpu.make_async_copy(node_tab.at[idx_n.at[s]], bn[b], sem_g[b]),
            pltpu.make_async_copy(in_tab.at[idx_i.at[s]], bi[b], sem_g[b]),
            pltpu.make_async_copy(out_tab.at[idx_o.at[s]], bo[b], sem_g[b]),
        )

    def wb_desc(c, b):
        return pltpu.make_async_copy(
            bn[b], out_hbm.at[pl.ds(w_base + c * CHUNK, CHUNK)], sem_w)

    for c0 in range(NSETS - 1):
        for d in gather_descs(c0, c0):
            d.start()

    @pl.loop(0, N_CHUNKS, step=NSETS)
    def _quad(cc):
        for b in range(NSETS):
            c = cc + b
            nb = (b + NSETS - 1) % NSETS
            if b == 0:
                @pl.when(cc > 0)
                def _():
                    wb_desc(cc - 1, nb).wait()
                for d in gather_descs(c + NSETS - 1, nb):
                    d.start()
            else:
                wb_desc(c - 1, nb).wait()

                @pl.when(c + NSETS - 1 < N_CHUNKS)
                def _():
                    for d in gather_descs(c + NSETS - 1, nb):
                        d.start()
            for d in gather_descs(c, b):
                d.wait()

            @pl.loop(0, CHUNK)
            def _row(r):
                for k in range(VREGS_PER_ROW):
                    sl = pl.ds(k * L, L)
                    bn[b][r, sl] = bn[b][r, sl] + bi[b][r, sl] + bo[b][r, sl]

            wb_desc(c, b).start()

    wb_desc(N_CHUNKS - 1, (N_CHUNKS - 1) % NSETS).wait()


@jax.jit
def _run(nt, ind, outd, node_tab, in_tab, out_tab):
    mesh = plsc.VectorSubcoreMesh(
        core_axis_name="c", subcore_axis_name="s", num_cores=NC,
        num_subcores=NS)

    def body(nt_hbm, in_hbm, ot_hbm, a, b_, c_, out_hbm,
             idx_n, idx_i, idx_o, *rest):
        return _sc_kernel(nt_hbm, in_hbm, ot_hbm, a, b_, c_, out_hbm,
                          idx_n, idx_i, idx_o, rest)

    f = pl.kernel(
        body,
        out_type=jax.ShapeDtypeStruct((R_TOTAL, EMBED), jnp.float32),
        mesh=mesh,
        scratch_types=(
            [pltpu.VMEM((ROWS_PER_W,), jnp.int32)] * 3
            + [pltpu.VMEM((CHUNK, EMBED), jnp.float32)] * (3 * NSETS)
            + [pltpu.SemaphoreType.DMA] * (NSETS + 1)
        ),
    )
    return f(nt, ind, outd, node_tab, in_tab, out_tab)


def kernel(node_type, in_degree, out_degree, node_table, in_degree_table,
           out_degree_table):
    n_graph, n_node = in_degree.shape
    nt = node_type.reshape(-1).astype(jnp.int32)
    ind = in_degree.reshape(-1).astype(jnp.int32)
    outd = out_degree.reshape(-1).astype(jnp.int32)
    out = _run(nt, ind, outd, node_table, in_degree_table, out_degree_table)
    return out.reshape(n_graph, n_node, EMBED)


# speedup vs baseline: 1.9921x; 1.0819x over previous
import jax
import jax.numpy as jnp
from jax import lax
from jax.experimental import pallas as pl
from jax.experimental.pallas import tpu as pltpu
from jax.experimental.pallas import tpu_sc as plsc

NC = 2
NS = 16
NW = NC * NS
L = 16

EMBED = 768
R_TOTAL = 1024 * 128
ROWS_PER_W = R_TOTAL // NW
CHUNK = 16
N_CHUNKS = ROWS_PER_W // CHUNK
VREGS_PER_ROW = EMBED // L


def _sc_kernel(nt_hbm, in_hbm, ot_hbm, node_tab, in_tab, out_tab, out_hbm,
               idx_n, idx_i, idx_o,
               bn0, bi0, bo0, bn1, bi1, bo1, sem_g0, sem_g1, sem_w):
    wid = lax.axis_index("s") * NC + lax.axis_index("c")
    w_base = wid * ROWS_PER_W

    bn = (bn0, bn1)
    bi = (bi0, bi1)
    bo = (bo0, bo1)
    sem_g = (sem_g0, sem_g1)

    pltpu.sync_copy(nt_hbm.at[pl.ds(w_base, ROWS_PER_W)], idx_n)
    pltpu.sync_copy(in_hbm.at[pl.ds(w_base, ROWS_PER_W)], idx_i)
    pltpu.sync_copy(ot_hbm.at[pl.ds(w_base, ROWS_PER_W)], idx_o)

    H = CHUNK // 2

    def gather_descs(c, b):
        descs = []
        for (tab, idx, buf) in ((node_tab, idx_n, bn[b]),
                                (in_tab, idx_i, bi[b]),
                                (out_tab, idx_o, bo[b])):
            for h in range(2):
                s = pl.ds(c * CHUNK + h * H, H)
                descs.append(pltpu.make_async_copy(
                    tab.at[idx.at[s]], buf.at[pl.ds(h * H, H)], sem_g[b]))
        return descs

    def wb_desc(c, b):
        return pltpu.make_async_copy(
            bn[b], out_hbm.at[pl.ds(w_base + c * CHUNK, CHUNK)], sem_w)

    for d in gather_descs(0, 0):
        d.start()

    @pl.loop(0, N_CHUNKS, step=2)
    def _pair(cc):
        for b in range(2):
            c = cc + b
            nb = 1 - b
            if b == 0:
                @pl.when(cc > 0)
                def _():
                    wb_desc(cc - 1, nb).wait()
                for d in gather_descs(c + 1, nb):
                    d.start()
            else:
                wb_desc(c - 1, nb).wait()

                @pl.when(cc < N_CHUNKS - 2)
                def _():
                    for d in gather_descs(c + 1, nb):
                        d.start()
            for d in gather_descs(c, b):
                d.wait()

            @pl.loop(0, CHUNK)
            def _row(r):
                for k in range(VREGS_PER_ROW):
                    sl = pl.ds(k * L, L)
                    bn[b][r, sl] = bn[b][r, sl] + bi[b][r, sl] + bo[b][r, sl]

            wb_desc(c, b).start()

    wb_desc(N_CHUNKS - 1, 1).wait()


@jax.jit
def _run(nt, ind, outd, node_tab, in_tab, out_tab):
    mesh = plsc.VectorSubcoreMesh(
        core_axis_name="c", subcore_axis_name="s", num_cores=NC,
        num_subcores=NS)
    f = pl.kernel(
        _sc_kernel,
        out_type=jax.ShapeDtypeStruct((R_TOTAL, EMBED), jnp.float32),
        mesh=mesh,
        scratch_types=[
            pltpu.VMEM((ROWS_PER_W,), jnp.int32),
            pltpu.VMEM((ROWS_PER_W,), jnp.int32),
            pltpu.VMEM((ROWS_PER_W,), jnp.int32),
            pltpu.VMEM((CHUNK, EMBED), jnp.float32),
            pltpu.VMEM((CHUNK, EMBED), jnp.float32),
            pltpu.VMEM((CHUNK, EMBED), jnp.float32),
            pltpu.VMEM((CHUNK, EMBED), jnp.float32),
            pltpu.VMEM((CHUNK, EMBED), jnp.float32),
            pltpu.VMEM((CHUNK, EMBED), jnp.float32),
            pltpu.SemaphoreType.DMA,
            pltpu.SemaphoreType.DMA,
            pltpu.SemaphoreType.DMA,
        ],
    )
    return f(nt, ind, outd, node_tab, in_tab, out_tab)


def kernel(node_type, in_degree, out_degree, node_table, in_degree_table,
           out_degree_table):
    n_graph, n_node = in_degree.shape
    nt = node_type.reshape(-1).astype(jnp.int32)
    ind = in_degree.reshape(-1).astype(jnp.int32)
    outd = out_degree.reshape(-1).astype(jnp.int32)
    out = _run(nt, ind, outd, node_table, in_degree_table, out_degree_table)
    return out.reshape(n_graph, n_node, EMBED)


# speedup vs baseline: 2.0462x; 1.0272x over previous
import jax
import jax.numpy as jnp
from jax import lax
from jax.experimental import pallas as pl
from jax.experimental.pallas import tpu as pltpu
from jax.experimental.pallas import tpu_sc as plsc

NC = 2
NS = 16
NW = NC * NS
L = 16

EMBED = 768
R_TOTAL = 1024 * 128
ROWS_PER_W = R_TOTAL // NW
CHUNK = 16
N_CHUNKS = ROWS_PER_W // CHUNK
VREGS_PER_ROW = EMBED // L


def _sc_kernel(nt_hbm, in_hbm, ot_hbm, node_tab, in_tab, out_tab, out_hbm,
               idx_n, idx_i, idx_o,
               bn0, bi0, bo0, bn1, bi1, bo1, sem_g0, sem_g1, sem_w):
    wid = lax.axis_index("s") * NC + lax.axis_index("c")
    w_base = wid * ROWS_PER_W

    bn = (bn0, bn1)
    bi = (bi0, bi1)
    bo = (bo0, bo1)
    sem_g = (sem_g0, sem_g1)

    pltpu.sync_copy(nt_hbm.at[pl.ds(w_base, ROWS_PER_W)], idx_n)
    pltpu.sync_copy(in_hbm.at[pl.ds(w_base, ROWS_PER_W)], idx_i)
    pltpu.sync_copy(ot_hbm.at[pl.ds(w_base, ROWS_PER_W)], idx_o)

    def gather_descs(c, b):
        s = pl.ds(c * CHUNK, CHUNK)
        return (
            pltpu.make_async_copy(node_tab.at[idx_n.at[s]], bn[b], sem_g[b]),
            pltpu.make_async_copy(in_tab.at[idx_i.at[s]], bi[b], sem_g[b]),
            pltpu.make_async_copy(out_tab.at[idx_o.at[s]], bo[b], sem_g[b]),
        )

    def wb_desc(c, b):
        return pltpu.make_async_copy(
            bn[b], out_hbm.at[pl.ds(w_base + c * CHUNK, CHUNK)], sem_w)

    for d in gather_descs(0, 0):
        d.start()

    @pl.loop(0, N_CHUNKS, step=2)
    def _pair(cc):
        for b in range(2):
            c = cc + b
            nb = 1 - b
            if b == 0:
                @pl.when(cc > 0)
                def _():
                    wb_desc(cc - 1, nb).wait()
                for d in gather_descs(c + 1, nb):
                    d.start()
            else:
                wb_desc(c - 1, nb).wait()

                @pl.when(cc < N_CHUNKS - 2)
                def _():
                    for d in gather_descs(c + 1, nb):
                        d.start()
            for d in gather_descs(c, b):
                d.wait()

            @pl.loop(0, CHUNK)
            def _row(r):
                for k in range(VREGS_PER_ROW):
                    sl = pl.ds(k * L, L)
                    bn[b][r, sl] = bn[b][r, sl] + bi[b][r, sl] + bo[b][r, sl]

            wb_desc(c, b).start()

    wb_desc(N_CHUNKS - 1, 1).wait()


@jax.jit
def _run(nt, ind, outd, node_tab, in_tab, out_tab):
    mesh = plsc.VectorSubcoreMesh(
        core_axis_name="c", subcore_axis_name="s", num_cores=NC,
        num_subcores=NS)
    f = pl.kernel(
        _sc_kernel,
        out_type=jax.ShapeDtypeStruct((R_TOTAL, EMBED), jnp.float32),
        mesh=mesh,
        scratch_types=[
            pltpu.VMEM((ROWS_PER_W,), jnp.int32),
            pltpu.VMEM((ROWS_PER_W,), jnp.int32),
            pltpu.VMEM((ROWS_PER_W,), jnp.int32),
            pltpu.VMEM((CHUNK, EMBED), jnp.float32),
            pltpu.VMEM((CHUNK, EMBED), jnp.float32),
            pltpu.VMEM((CHUNK, EMBED), jnp.float32),
            pltpu.VMEM((CHUNK, EMBED), jnp.float32),
            pltpu.VMEM((CHUNK, EMBED), jnp.float32),
            pltpu.VMEM((CHUNK, EMBED), jnp.float32),
            pltpu.SemaphoreType.DMA,
            pltpu.SemaphoreType.DMA,
            pltpu.SemaphoreType.DMA,
        ],
    )
    return f(nt, ind, outd, node_tab, in_tab, out_tab)


def kernel(node_type, in_degree, out_degree, node_table, in_degree_table,
           out_degree_table):
    n_graph, n_node = in_degree.shape
    nt = node_type.reshape(-1).astype(jnp.int32)
    ind = in_degree.reshape(-1).astype(jnp.int32)
    outd = out_degree.reshape(-1).astype(jnp.int32)
    out = _run(nt, ind, outd, node_table, in_degree_table, out_degree_table)
    return out.reshape(n_graph, n_node, EMBED)


# speedup vs baseline: 2.0868x; 1.0198x over previous
import jax
import jax.numpy as jnp
from jax import lax
from jax.experimental import pallas as pl
from jax.experimental.pallas import tpu as pltpu
from jax.experimental.pallas import tpu_sc as plsc

NC = 2
NS = 16
NW = NC * NS
L = 16

EMBED = 768
R_TOTAL = 1024 * 128
ROWS_PER_W = R_TOTAL // NW
CHUNK = 16
NSETS = 3
N_CHUNKS = ROWS_PER_W // CHUNK
N_MAIN = (N_CHUNKS - 1) // NSETS * NSETS
VREGS_PER_ROW = EMBED // L


def _sc_kernel(nt_hbm, in_hbm, ot_hbm, node_tab, in_tab, out_tab, out_hbm,
               idx_n, idx_i, idx_o,
               bn0, bi0, bo0, bn1, bi1, bo1, bn2, bi2, bo2,
               sg0, sg1, sg2, sem_w):
    wid = lax.axis_index("s") * NC + lax.axis_index("c")
    w_base = wid * ROWS_PER_W

    bn = (bn0, bn1, bn2)
    bi = (bi0, bi1, bi2)
    bo = (bo0, bo1, bo2)
    sem_g = (sg0, sg1, sg2)

    pltpu.sync_copy(nt_hbm.at[pl.ds(w_base, ROWS_PER_W)], idx_n)
    pltpu.sync_copy(in_hbm.at[pl.ds(w_base, ROWS_PER_W)], idx_i)
    pltpu.sync_copy(ot_hbm.at[pl.ds(w_base, ROWS_PER_W)], idx_o)

    def gather_descs(c, b):
        s = pl.ds(c * CHUNK, CHUNK)
        return (
            pltpu.make_async_copy(node_tab.at[idx_n.at[s]], bn[b], sem_g[b]),
            pltpu.make_async_copy(in_tab.at[idx_i.at[s]], bi[b], sem_g[b]),
            pltpu.make_async_copy(out_tab.at[idx_o.at[s]], bo[b], sem_g[b]),
        )

    def wb_desc(c, b):
        return pltpu.make_async_copy(
            bn[b], out_hbm.at[pl.ds(w_base + c * CHUNK, CHUNK)], sem_w)

    def consume(c, b):
        for d in gather_descs(c, b):
            d.wait()

        @pl.loop(0, CHUNK)
        def _row(r):
            for k in range(VREGS_PER_ROW):
                sl = pl.ds(k * L, L)
                bn[b][r, sl] = bn[b][r, sl] + bi[b][r, sl] + bo[b][r, sl]

        wb_desc(c, b).start()

    for c0 in range(NSETS - 1):
        for d in gather_descs(c0, c0):
            d.start()

    @pl.loop(0, N_MAIN, step=NSETS)
    def _triple(cc):
        for b in range(NSETS):
            c = cc + b
            nb = (b + NSETS - 1) % NSETS
            if b == 0:
                @pl.when(cc > 0)
                def _():
                    wb_desc(cc - 1, nb).wait()
            else:
                wb_desc(c - 1, nb).wait()

            @pl.when(c + NSETS - 1 < N_CHUNKS)
            def _():
                for d in gather_descs(c + NSETS - 1, nb):
                    d.start()
            consume(c, b)

    for c in range(N_MAIN, N_CHUNKS):
        wb_desc(c - 1, (c - 1) % NSETS).wait()
        consume(c, c % NSETS)
    wb_desc(N_CHUNKS - 1, (N_CHUNKS - 1) % NSETS).wait()


@jax.jit
def _run(nt, ind, outd, node_tab, in_tab, out_tab):
    mesh = plsc.VectorSubcoreMesh(
        core_axis_name="c", subcore_axis_name="s", num_cores=NC,
        num_subcores=NS)
    f = pl.kernel(
        _sc_kernel,
        out_type=jax.ShapeDtypeStruct((R_TOTAL, EMBED), jnp.float32),
        mesh=mesh,
        scratch_types=(
            [pltpu.VMEM((ROWS_PER_W,), jnp.int32)] * 3
            + [pltpu.VMEM((CHUNK, EMBED), jnp.float32)] * (3 * NSETS)
            + [pltpu.SemaphoreType.DMA] * (NSETS + 1)
        ),
    )
    return f(nt, ind, outd, node_tab, in_tab, out_tab)


def kernel(node_type, in_degree, out_degree, node_table, in_degree_table,
           out_degree_table):
    n_graph, n_node = in_degree.shape
    nt = node_type.reshape(-1).astype(jnp.int32)
    ind = in_degree.reshape(-1).astype(jnp.int32)
    outd = out_degree.reshape(-1).astype(jnp.int32)
    out = _run(nt, ind, outd, node_table, in_degree_table, out_degree_table)
    return out.reshape(n_graph, n_node, EMBED)


# speedup vs baseline: 2.5661x; 1.2297x over previous
import jax
import jax.numpy as jnp
from jax import lax
from jax.experimental import pallas as pl
from jax.experimental.pallas import tpu as pltpu
from jax.experimental.pallas import tpu_sc as plsc

NC = 2
NS = 16
NW = NC * NS
L = 16

EMBED = 768
R_TOTAL = 1024 * 128
ROWS_PER_W = R_TOTAL // NW
CHUNK = 16
NSETS = 3
N_CHUNKS = ROWS_PER_W // CHUNK
N_MAIN = (N_CHUNKS - 1) // NSETS * NSETS
VREGS_PER_ROW = EMBED // L


def _sc_kernel(nt_hbm, in_hbm, ot_hbm, node_tab, in_tab, out_tab, out_hbm,
               idx_n, idx_i, idx_o,
               bn0, bi0, bo0, bn1, bi1, bo1, bn2, bi2, bo2,
               sg0, sg1, sg2, sem_w):
    wid = lax.axis_index("s") * NC + lax.axis_index("c")
    w_base = wid * ROWS_PER_W

    bn = (bn0, bn1, bn2)
    bi = (bi0, bi1, bi2)
    bo = (bo0, bo1, bo2)
    sem_g = (sg0, sg1, sg2)

    pltpu.sync_copy(nt_hbm.at[pl.ds(w_base, ROWS_PER_W)], idx_n)
    pltpu.sync_copy(in_hbm.at[pl.ds(w_base, ROWS_PER_W)], idx_i)
    pltpu.sync_copy(ot_hbm.at[pl.ds(w_base, ROWS_PER_W)], idx_o)

    def gather_descs(c, b):
        s = pl.ds(c * CHUNK, CHUNK)
        return (
            pltpu.make_async_copy(node_tab.at[idx_n.at[s]], bn[b], sem_g[b]),
            pltpu.make_async_copy(in_tab.at[idx_i.at[s]], bi[b], sem_g[b]),
            pltpu.make_async_copy(out_tab.at[idx_o.at[s]], bo[b], sem_g[b]),
        )

    def wb_desc(c, b):
        return pltpu.make_async_copy(
            bn[b], out_hbm.at[pl.ds(w_base + c * CHUNK, CHUNK)], sem_w)

    def consume(c, b):
        for d in gather_descs(c, b):
            d.wait()

        @plsc.parallel_loop(0, VREGS_PER_ROW * L, step=L)
        def _col(k):
            sl = pl.ds(k, L)
            for r in range(CHUNK):
                bn[b][r, sl] = bn[b][r, sl] + bi[b][r, sl] + bo[b][r, sl]

        wb_desc(c, b).start()

    for c0 in range(NSETS - 1):
        for d in gather_descs(c0, c0):
            d.start()

    @pl.loop(0, N_MAIN, step=NSETS)
    def _triple(cc):
        for b in range(NSETS):
            c = cc + b
            nb = (b + NSETS - 1) % NSETS
            if b == 0:
                @pl.when(cc > 0)
                def _():
                    wb_desc(cc - 1, nb).wait()
            else:
                wb_desc(c - 1, nb).wait()

            @pl.when(c + NSETS - 1 < N_CHUNKS)
            def _():
                for d in gather_descs(c + NSETS - 1, nb):
                    d.start()
            consume(c, b)

    for c in range(N_MAIN, N_CHUNKS):
        wb_desc(c - 1, (c - 1) % NSETS).wait()
        consume(c, c % NSETS)
    wb_desc(N_CHUNKS - 1, (N_CHUNKS - 1) % NSETS).wait()


@jax.jit
def _run(nt, ind, outd, node_tab, in_tab, out_tab):
    mesh = plsc.VectorSubcoreMesh(
        core_axis_name="c", subcore_axis_name="s", num_cores=NC,
        num_subcores=NS)
    f = pl.kernel(
        _sc_kernel,
        out_type=jax.ShapeDtypeStruct((R_TOTAL, EMBED), jnp.float32),
        mesh=mesh,
        scratch_types=(
            [pltpu.VMEM((ROWS_PER_W,), jnp.int32)] * 3
            + [pltpu.VMEM((CHUNK, EMBED), jnp.float32)] * (3 * NSETS)
            + [pltpu.SemaphoreType.DMA] * (NSETS + 1)
        ),
    )
    return f(nt, ind, outd, node_tab, in_tab, out_tab)


def kernel(node_type, in_degree, out_degree, node_table, in_degree_table,
           out_degree_table):
    n_graph, n_node = in_degree.shape
    nt = node_type.reshape(-1).astype(jnp.int32)
    ind = in_degree.reshape(-1).astype(jnp.int32)
    outd = out_degree.reshape(-1).astype(jnp.int32)
    out = _run(nt, ind, outd, node_table, in_degree_table, out_degree_table)
    return out.reshape(n_graph, n_node, EMBED)
